# baseline (speedup 1.0000x reference)
import functools

import jax
import jax.numpy as jnp
from jax import lax
from jax.experimental import pallas as pl
from jax.experimental.pallas import tpu as pltpu
from jax.experimental.pallas import tpu_sc as plsc

H = 128
HW = 64
NUM_REL = 5
NC = 2
NS = 16
NW = NC * NS
CB = 80
DC = 16
NSUB = CB // 16
NLB = NUM_REL - 1


def _decode(z_user, z_movie, rel_emb, src3, dst3):
    nw, npw, cb = src3.shape
    assert cb == CB and nw == NW and npw % 3 == 2
    ntrip = npw // 3
    mesh = plsc.VectorSubcoreMesh(core_axis_name="c", subcore_axis_name="s",
                                  num_cores=NC, num_subcores=NS)

    @functools.partial(
        pl.kernel,
        out_type=jax.ShapeDtypeStruct((NW, npw, CB), jnp.float32),
        mesh=mesh,
        compiler_params=pltpu.CompilerParams(needs_layout_passes=False,
                                            use_tc_tiling_on_sc=False),
        scratch_types=[
            pltpu.VMEM((NUM_REL - 1, HW), jnp.int32),
            pltpu.VMEM((npw, CB), jnp.int32),
            pltpu.VMEM((npw, CB), jnp.int32),
            pltpu.VMEM((CB, HW), jnp.int32),
            pltpu.VMEM((CB, HW), jnp.int32),
            pltpu.VMEM((CB, HW), jnp.int32),
            pltpu.VMEM((CB, HW), jnp.int32),
            pltpu.VMEM((CB, HW), jnp.int32),
            pltpu.VMEM((CB, HW), jnp.int32),
            pltpu.VMEM((npw, CB), jnp.float32),
            pltpu.SemaphoreType.DMA,
            pltpu.SemaphoreType.DMA,
            pltpu.SemaphoreType.DMA,
        ],
    )
    def decode_k(zu, zm, rel, src, dst, out, rel_v, si_v, di_v,
                 uA, mA, uB, mB, uC, mC, o_v, semA, semB, semC):
        wid = lax.axis_index("s") * NC + lax.axis_index("c")
        pltpu.sync_copy(rel, rel_v)
        pltpu.sync_copy(src.at[wid], si_v)
        pltpu.sync_copy(dst.at[wid], di_v)

        iota = lax.iota(jnp.int32, 16)
        row_idx = [iota + (g * 16) for g in range(NSUB)]
        zero = jnp.zeros((16,), jnp.float32)

        def issue(c, u_buf, m_buf, sem):
            pltpu.async_copy(zu.at[si_v.at[c]], u_buf, sem)
            pltpu.async_copy(zm.at[di_v.at[c]], m_buf, sem)

        def drain(c, u_buf, m_buf, sem):
            pltpu.make_async_copy(zu.at[si_v.at[c]], u_buf, sem).wait()
            pltpu.make_async_copy(zm.at[di_v.at[c]], m_buf, sem).wait()

        def compute(c, u_buf, m_buf):
            init = tuple(tuple(zero for _ in range(NLB))
                         for _ in range(NSUB))
            bzero = jnp.zeros((2 * DC,), jnp.bfloat16)

            @pl.loop(0, HW // DC, init_carry=init)
            def acc_loop(dc, acc):
                acc = [list(a) for a in acc]
                accb = [[bzero for _ in range(NLB)] for _ in range(NSUB)]
                col_base = jnp.full((16,), dc * DC, jnp.int32)
                rv = [rel_v[l % (NUM_REL - 1), pl.ds(dc * DC, DC)]
                      for l in range(NLB)]
                for j in range(DC):
                    shift = (j + iota) % DC
                    col = col_base + shift
                    rb = [plsc.bitcast(
                              rv[l].at[shift].get(mode="promise_in_bounds"),
                              jnp.bfloat16)
                          for l in range(NLB)]
                    for g in range(NSUB):
                        u = plsc.bitcast(
                            plsc.load_gather(u_buf, [row_idx[g], col]),
                            jnp.bfloat16)
                        m = plsc.bitcast(
                            plsc.load_gather(m_buf, [row_idx[g], col]),
                            jnp.bfloat16)
                        s = u * m
                        for l in range(NLB):
                            accb[g][l] = accb[g][l] + s * rb[l]
                for g in range(NSUB):
                    for l in range(NLB):
                        lo, hi = plsc.unpack(
                            accb[g][l], format=plsc.PackFormat.INTERLEAVED,
                            preferred_element_type=jnp.float32)
                        acc[g][l] = acc[g][l] + lo + hi
                return tuple(tuple(a) for a in acc)

            for g in range(NSUB):
                a = acc_loop[g]
                mx = zero
                for l in range(NLB):
                    mx = jnp.maximum(mx, a[l])
                e0 = jnp.exp(zero - mx)
                e = [jnp.exp(a[l] - mx) for l in range(NLB)]
                den = e0
                num = zero
                for l in range(NLB):
                    den = den + e[l]
                    num = num + jnp.float32(l + 1) * e[l]
                o_v[c, pl.ds(g * 16, 16)] = num / den

        issue(0, uA, mA, semA)
        issue(1, uB, mB, semB)

        @pl.loop(0, ntrip)
        def _trip(i):
            c0 = 3 * i
            issue(c0 + 2, uC, mC, semC)
            drain(c0, uA, mA, semA)
            compute(c0, uA, mA)
            issue(c0 + 3, uA, mA, semA)
            drain(c0 + 1, uB, mB, semB)
            compute(c0 + 1, uB, mB)
            issue(c0 + 4, uB, mB, semB)
            drain(c0 + 2, uC, mC, semC)
            compute(c0 + 2, uC, mC)

        drain(npw - 2, uA, mA, semA)
        compute(npw - 2, uA, mA)
        drain(npw - 1, uB, mB, semB)
        compute(npw - 1, uB, mB)

        pltpu.sync_copy(o_v, out.at[wid])

    return decode_k(z_user, z_movie, rel_emb, src3, dst3)


def _pack_bf16(x):
    n = x.shape[0]
    b = x.astype(jnp.bfloat16).reshape(n, HW, 2)
    return jax.lax.bitcast_convert_type(b, jnp.int32)


def kernel(z_user, z_movie, rel_emb, edge_label_index):
    E = edge_label_index.shape[1]
    npw = E // (NW * CB)
    src3 = edge_label_index[0].reshape(NW, npw, CB)
    dst3 = edge_label_index[1].reshape(NW, npw, CB)
    rel_shift = rel_emb[1:] - rel_emb[0:1]
    out3 = _decode(_pack_bf16(z_user), _pack_bf16(z_movie),
                   _pack_bf16(rel_shift), src3, dst3)
    return out3.reshape(E)


# speedup vs baseline: 1.1350x; 1.1350x over previous
import functools

import jax
import jax.numpy as jnp
from jax import lax
from jax.experimental import pallas as pl
from jax.experimental.pallas import tpu as pltpu
from jax.experimental.pallas import tpu_sc as plsc

H = 128
HW = 64
NUM_REL = 5
NC = 2
NS = 16
NW = NC * NS
CB = 80
DC = 16
NSUB = CB // 16
NLB = NUM_REL - 1


def _decode(z_user, z_movie, rel_emb, src3, dst3):
    nw, npw, cb = src3.shape
    assert cb == CB and nw == NW and npw % 2 == 1
    npairs = npw // 2
    mesh = plsc.VectorSubcoreMesh(core_axis_name="c", subcore_axis_name="s",
                                  num_cores=NC, num_subcores=NS)

    @functools.partial(
        pl.kernel,
        out_type=jax.ShapeDtypeStruct((NW, npw, CB), jnp.float32),
        mesh=mesh,
        compiler_params=pltpu.CompilerParams(needs_layout_passes=False,
                                            use_tc_tiling_on_sc=False),
        scratch_types=[
            pltpu.VMEM((NUM_REL - 1, HW), jnp.int32),
            pltpu.VMEM((npw, CB), jnp.int32),
            pltpu.VMEM((npw, CB), jnp.int32),
            pltpu.VMEM((CB, HW), jnp.int32),
            pltpu.VMEM((CB, HW), jnp.int32),
            pltpu.VMEM((CB, HW), jnp.int32),
            pltpu.VMEM((CB, HW), jnp.int32),
            pltpu.VMEM((npw, CB), jnp.float32),
            pltpu.SemaphoreType.DMA,
            pltpu.SemaphoreType.DMA,
        ],
    )
    def decode_k(zu, zm, rel, src, dst, out, rel_v, si_v, di_v,
                 uA, mA, uB, mB, o_v, semA, semB):
        wid = lax.axis_index("s") * NC + lax.axis_index("c")
        pltpu.sync_copy(rel, rel_v)
        pltpu.sync_copy(src.at[wid], si_v)
        pltpu.sync_copy(dst.at[wid], di_v)

        iota = lax.iota(jnp.int32, 16)
        row_idx = [iota + (g * 16) for g in range(NSUB)]
        zero = jnp.zeros((16,), jnp.float32)

        def issue(c, u_buf, m_buf, sem):
            pltpu.async_copy(zu.at[si_v.at[c]], u_buf, sem)
            pltpu.async_copy(zm.at[di_v.at[c]], m_buf, sem)

        def drain(c, u_buf, m_buf, sem):
            pltpu.make_async_copy(zu.at[si_v.at[c]], u_buf, sem).wait()
            pltpu.make_async_copy(zm.at[di_v.at[c]], m_buf, sem).wait()

        def compute(c, u_buf, m_buf):
            init = tuple(tuple(zero for _ in range(NLB))
                         for _ in range(NSUB))
            bzero = jnp.zeros((2 * DC,), jnp.bfloat16)

            @pl.loop(0, HW // DC, init_carry=init)
            def acc_loop(dc, acc):
                acc = [list(a) for a in acc]
                accb = [[bzero for _ in range(NLB)] for _ in range(NSUB)]
                col_base = jnp.full((16,), dc * DC, jnp.int32)
                rv = [rel_v[l % (NUM_REL - 1), pl.ds(dc * DC, DC)]
                      for l in range(NLB)]
                for j in range(DC):
                    shift = (j + iota) % DC
                    col = col_base + shift
                    rb = [plsc.bitcast(
                              rv[l].at[shift].get(mode="promise_in_bounds"),
                              jnp.bfloat16)
                          for l in range(NLB)]
                    for g in range(NSUB):
                        u = plsc.bitcast(
                            plsc.load_gather(u_buf, [row_idx[g], col]),
                            jnp.bfloat16)
                        m = plsc.bitcast(
                            plsc.load_gather(m_buf, [row_idx[g], col]),
                            jnp.bfloat16)
                        s = u * m
                        for l in range(NLB):
                            accb[g][l] = accb[g][l] + s * rb[l]
                for g in range(NSUB):
                    for l in range(NLB):
                        lo, hi = plsc.unpack(
                            accb[g][l], format=plsc.PackFormat.INTERLEAVED,
                            preferred_element_type=jnp.float32)
                        acc[g][l] = acc[g][l] + lo + hi
                return tuple(tuple(a) for a in acc)

            for g in range(NSUB):
                a = acc_loop[g]
                mx = zero
                for l in range(NLB):
                    mx = jnp.maximum(mx, a[l])
                e0 = jnp.exp(zero - mx)
                e = [jnp.exp(a[l] - mx) for l in range(NLB)]
                den = e0
                num = zero
                for l in range(NLB):
                    den = den + e[l]
                    num = num + jnp.float32(l + 1) * e[l]
                o_v[c, pl.ds(g * 16, 16)] = num / den

        issue(0, uA, mA, semA)

        @pl.loop(0, npairs)
        def _pair(i):
            cA = 2 * i
            cB = cA + 1
            issue(cB, uB, mB, semB)
            drain(cA, uA, mA, semA)
            compute(cA, uA, mA)
            issue(cA + 2, uA, mA, semA)
            drain(cB, uB, mB, semB)
            compute(cB, uB, mB)

        last = npw - 1
        drain(last, uA, mA, semA)
        compute(last, uA, mA)

        pltpu.sync_copy(o_v, out.at[wid])

    return decode_k(z_user, z_movie, rel_emb, src3, dst3)


def _pack_bf16(x):
    n = x.shape[0]
    b = x.astype(jnp.bfloat16).reshape(n, HW, 2)
    return jax.lax.bitcast_convert_type(b, jnp.int32)


def kernel(z_user, z_movie, rel_emb, edge_label_index):
    E = edge_label_index.shape[1]
    npw = E // (NW * CB)
    src3 = edge_label_index[0].reshape(NW, npw, CB)
    dst3 = edge_label_index[1].reshape(NW, npw, CB)
    rel_shift = rel_emb[1:] - rel_emb[0:1]
    out3 = _decode(_pack_bf16(z_user), _pack_bf16(z_movie),
                   _pack_bf16(rel_shift), src3, dst3)
    return out3.reshape(E)
